# baseline (speedup 1.0000x reference)
import functools

import jax
import jax.numpy as jnp
from jax import lax
from jax.experimental import pallas as pl
from jax.experimental.pallas import tpu as pltpu
from jax.experimental.pallas import tpu_sc as plsc

N_NODES = 10000
N_PAD = 10240
N_EDGES = 320000
N_TILES = 16
B_EDGES = 112
EPT = 20608
N_CHUNKS = EPT // B_EDGES
STAGE_ROWS = 8
N_STAGES = N_CHUNKS // STAGE_ROWS
NBUF = 3
ROWS_PER_TILE = N_PAD // N_TILES


def _mm_body(x_ref, w_ref, o_ref, *, relu_in):
    x = x_ref[...]
    if relu_in:
        x = jnp.maximum(x, 0.0)
    o_ref[...] = jnp.dot(x, w_ref[...], preferred_element_type=jnp.float32)


def _matmul(x, w, relu_in=False, bm=1000):
    n, k = x.shape
    f = w.shape[1]
    return pl.pallas_call(
        functools.partial(_mm_body, relu_in=relu_in),
        grid=(n // bm,),
        in_specs=[
            pl.BlockSpec((bm, k), lambda i: (i, 0)),
            pl.BlockSpec((k, f), lambda i: (0, 0)),
        ],
        out_specs=pl.BlockSpec((bm, f), lambda i: (i, 0)),
        out_shape=jax.ShapeDtypeStruct((n, f), jnp.float32),
    )(x, w)


def _norm_body(x_ref, o_ref):
    y = jnp.maximum(x_ref[...], 0.0)
    nrm = jnp.sqrt(jnp.sum(y * y, axis=1, keepdims=True))
    o_ref[...] = y / jnp.maximum(nrm, 1e-12)


def _relu_l2norm(x, bm=1000):
    n, f = x.shape
    return pl.pallas_call(
        _norm_body,
        grid=(n // bm,),
        in_specs=[pl.BlockSpec((bm, f), lambda i: (i, 0))],
        out_specs=pl.BlockSpec((bm, f), lambda i: (i, 0)),
        out_shape=jax.ShapeDtypeStruct((n, f), jnp.float32),
    )(x)


def _relu_body(x_ref, o_ref):
    o_ref[...] = jnp.maximum(x_ref[...], 0.0)


def _relu(x, bm=1000):
    n, f = x.shape
    return pl.pallas_call(
        _relu_body,
        grid=(n // bm,),
        in_specs=[pl.BlockSpec((bm, f), lambda i: (i, 0))],
        out_specs=pl.BlockSpec((bm, f), lambda i: (i, 0)),
        out_shape=jax.ShapeDtypeStruct((n, f), jnp.float32),
    )(x)


def _sc_spmm(sup_sr, sup_tg, esr, etg, feat, tc_tiling=True):
    mesh = plsc.VectorSubcoreMesh(core_axis_name="c", subcore_axis_name="s")
    nvec = feat // 16
    cparams = (None if tc_tiling
               else pltpu.CompilerParams(use_tc_tiling_on_sc=False))

    @functools.partial(
        pl.kernel,
        mesh=mesh,
        out_type=(
            jax.ShapeDtypeStruct((N_PAD, feat), jnp.float32),
            jax.ShapeDtypeStruct((N_PAD, feat), jnp.float32),
        ),
        scratch_types=[
            pltpu.VMEM((STAGE_ROWS, B_EDGES), jnp.int32),
            pltpu.VMEM((STAGE_ROWS, B_EDGES), jnp.int32),
            pltpu.VMEM((STAGE_ROWS, B_EDGES), jnp.float32),
            pltpu.VMEM((B_EDGES, feat), jnp.float32),
            pltpu.VMEM((B_EDGES, feat), jnp.float32),
            pltpu.VMEM((B_EDGES, feat), jnp.float32),
            pltpu.VMEM_SHARED((N_PAD, feat), jnp.float32),
            pltpu.SemaphoreType.DMA,
            pltpu.SemaphoreType.DMA,
            pltpu.SemaphoreType.DMA,
            pltpu.SemaphoreType.DMA,
            pltpu.SemaphoreType.DMA,
            pltpu.SemaphoreType.DMA,
        ],
        compiler_params=cparams,
    )
    def spmm(sup_sr_h, sup_tg_h, ssr_h, dsr_h, wsr_h, stg_h, dtg_h, wtg_h,
             out_sr_h, out_tg_h, src_v, dst_v, w_v, rows_a, rows_b, rows_c,
             acc, gsem_a, gsem_b, gsem_c, ssem_a, ssem_b, ssem_c):
        g = lax.axis_index("c")
        sid = lax.axis_index("s")
        bufs = (rows_a, rows_b, rows_c)
        gsems = (gsem_a, gsem_b, gsem_c)
        ssems = (ssem_a, ssem_b, ssem_c)

        def zrow(r, c):
            for j in range(nvec):
                rows_a[r, pl.ds(j * 16, 16)] = jnp.zeros((16,), jnp.float32)
            return c
        lax.fori_loop(0, B_EDGES, zrow, 0)
        base = sid * ROWS_PER_TILE
        for i in range(ROWS_PER_TILE // B_EDGES):
            pltpu.sync_copy(rows_a, acc.at[pl.ds(base + i * B_EDGES, B_EDGES)])
        rem = ROWS_PER_TILE % B_EDGES
        if rem:
            pltpu.sync_copy(
                rows_a.at[pl.ds(0, rem)],
                acc.at[pl.ds(base + ROWS_PER_TILE - rem, rem)])
        plsc.subcore_barrier()

        def run(sup_h, s_h, d_h, w_h, out_h):
            def stage(si, c):
                sl = pl.ds(si * STAGE_ROWS, STAGE_ROWS)
                pltpu.sync_copy(s_h.at[sid, sl], src_v)
                pltpu.sync_copy(d_h.at[sid, sl], dst_v)
                pltpu.sync_copy(w_h.at[sid, sl], w_v)

                gd = {
                    0: pltpu.async_copy(sup_h.at[src_v.at[0]], bufs[0],
                                        gsems[0]),
                    1: pltpu.async_copy(sup_h.at[src_v.at[1]], bufs[1],
                                        gsems[1]),
                }
                sd = {}
                for r in range(STAGE_ROWS):
                    b = r % NBUF
                    gd[r].wait()
                    if r + 2 < STAGE_ROWS:
                        nb = (r + 2) % NBUF
                        if r >= 1:
                            sd[r - 1].wait()
                        gd[r + 2] = pltpu.async_copy(
                            sup_h.at[src_v.at[r + 2]], bufs[nb], gsems[nb])

                    def edge_group(gi, c3, _r=r, _b=b):
                        wvec = w_v[_r, pl.ds(gi * 16, 16)]
                        for i in range(16):
                            wv = wvec[i]
                            e = gi * 16 + i
                            for j in range(nvec):
                                fl = pl.ds(j * 16, 16)
                                bufs[_b][e, fl] = bufs[_b][e, fl] * wv
                        return c3
                    lax.fori_loop(0, B_EDGES // 16, edge_group, 0)
                    sd[r] = pltpu.async_copy(bufs[b], acc.at[dst_v.at[r]],
                                             ssems[b], add=True)
                for r in range(STAGE_ROWS - 3, STAGE_ROWS):
                    sd[r].wait()
                return c
            lax.fori_loop(0, N_STAGES, stage, 0)
            plsc.subcore_barrier()
            pltpu.sync_copy(
                acc.at[pl.ds(sid * ROWS_PER_TILE, ROWS_PER_TILE)],
                out_h.at[pl.ds(sid * ROWS_PER_TILE, ROWS_PER_TILE)])

        @pl.when(g == 0)
        def _():
            run(sup_sr_h, ssr_h, dsr_h, wsr_h, out_sr_h)

        @pl.when(g == 1)
        def _():
            run(sup_tg_h, stg_h, dtg_h, wtg_h, out_tg_h)

    return spmm(sup_sr, sup_tg, *esr, *etg)


def kernel(edge_index_sr, edge_index_tg, edge_weight_sr, edge_weight_tg,
           attr_weight_sr, attr_weight_tg, emb_sr, emb_tg,
           W_s0, W_s1, W_a11, W_a12, W_a2):
    ept_raw = N_EDGES // N_TILES

    def prep(ei, ew):
        def shape_idx(a):
            a = jnp.asarray(a, jnp.int32).reshape(N_TILES, ept_raw)
            a = jnp.pad(a, ((0, 0), (0, EPT - ept_raw)))
            return a.reshape(N_TILES, N_CHUNKS, B_EDGES)
        src = shape_idx(ei[0])
        dst = shape_idx(ei[1])
        w = ew.astype(jnp.float32).reshape(N_TILES, ept_raw)
        w = jnp.pad(w, ((0, 0), (0, EPT - ept_raw)))
        w = w.reshape(N_TILES, N_CHUNKS, B_EDGES)
        return (src, dst, w)

    esr = prep(edge_index_sr, edge_weight_sr)
    etg = prep(edge_index_tg, edge_weight_tg)

    s_sr = _matmul(emb_sr, W_s0)
    s_tg = _matmul(emb_tg, W_s0)
    a1_sr, a1_tg = _sc_spmm(s_sr, s_tg, esr, etg, 128)
    s2_sr = _matmul(a1_sr, W_s1, relu_in=True, bm=1024)
    s2_tg = _matmul(a1_tg, W_s1, relu_in=True, bm=1024)
    a2_sr, a2_tg = _sc_spmm(s2_sr, s2_tg, esr, etg, 128)
    sr_s = _relu_l2norm(a2_sr[:N_NODES])
    tg_s = _relu_l2norm(a2_tg[:N_NODES])

    t_sr = _matmul(attr_weight_sr, W_a11)
    t_tg = _matmul(attr_weight_tg, W_a12)
    b1_sr, b1_tg = _sc_spmm(t_sr, t_tg, esr, etg, 64, tc_tiling=False)
    t2_sr = _matmul(b1_sr, W_a2, relu_in=True, bm=1024)
    t2_tg = _matmul(b1_tg, W_a2, relu_in=True, bm=1024)
    b2_sr, b2_tg = _sc_spmm(t2_sr, t2_tg, esr, etg, 64, tc_tiling=False)
    sr_a = _relu(b2_sr[:N_NODES])
    tg_a = _relu(b2_tg[:N_NODES])

    return (sr_s, tg_s, sr_a, tg_a)


# speedup vs baseline: 1.6660x; 1.6660x over previous
import functools

import jax
import jax.numpy as jnp
from jax import lax
from jax.experimental import pallas as pl
from jax.experimental.pallas import tpu as pltpu
from jax.experimental.pallas import tpu_sc as plsc

N_NODES = 10000
N_PAD = 10240
N_EDGES = 320000
N_TILES = 16
B_EDGES = 160
EPT = N_EDGES // N_TILES
N_CHUNKS = EPT // B_EDGES
STAGE_ROWS = 5
N_STAGES = N_CHUNKS // STAGE_ROWS
ROWS_PER_TILE = N_PAD // N_TILES


def _mm_body(x_ref, w_ref, o_ref, *, relu_in):
    x = x_ref[...]
    if relu_in:
        x = jnp.maximum(x, 0.0)
    o_ref[...] = jnp.dot(x, w_ref[...], preferred_element_type=jnp.float32)


def _matmul(x, w, relu_in=False, bm=1000):
    n, k = x.shape
    f = w.shape[1]
    return pl.pallas_call(
        functools.partial(_mm_body, relu_in=relu_in),
        grid=(n // bm,),
        in_specs=[
            pl.BlockSpec((bm, k), lambda i: (i, 0)),
            pl.BlockSpec((k, f), lambda i: (0, 0)),
        ],
        out_specs=pl.BlockSpec((bm, f), lambda i: (i, 0)),
        out_shape=jax.ShapeDtypeStruct((n, f), jnp.float32),
    )(x, w)


def _norm_body(x_ref, o_ref):
    y = jnp.maximum(x_ref[...], 0.0)
    nrm = jnp.sqrt(jnp.sum(y * y, axis=1, keepdims=True))
    o_ref[...] = y / jnp.maximum(nrm, 1e-12)


def _relu_l2norm(x, bm=1000):
    n, f = x.shape
    return pl.pallas_call(
        _norm_body,
        grid=(n // bm,),
        in_specs=[pl.BlockSpec((bm, f), lambda i: (i, 0))],
        out_specs=pl.BlockSpec((bm, f), lambda i: (i, 0)),
        out_shape=jax.ShapeDtypeStruct((n, f), jnp.float32),
    )(x)


def _relu_body(x_ref, o_ref):
    o_ref[...] = jnp.maximum(x_ref[...], 0.0)


def _relu(x, bm=1000):
    n, f = x.shape
    return pl.pallas_call(
        _relu_body,
        grid=(n // bm,),
        in_specs=[pl.BlockSpec((bm, f), lambda i: (i, 0))],
        out_specs=pl.BlockSpec((bm, f), lambda i: (i, 0)),
        out_shape=jax.ShapeDtypeStruct((n, f), jnp.float32),
    )(x)


def _sc_spmm(sup_sr, sup_tg, esr, etg, feat):
    mesh = plsc.VectorSubcoreMesh(core_axis_name="c", subcore_axis_name="s")
    nvec = feat // 16

    @functools.partial(
        pl.kernel,
        mesh=mesh,
        out_type=(
            jax.ShapeDtypeStruct((N_PAD, feat), jnp.float32),
            jax.ShapeDtypeStruct((N_PAD, feat), jnp.float32),
        ),
        scratch_types=[
            pltpu.VMEM((STAGE_ROWS, B_EDGES), jnp.int32),
            pltpu.VMEM((STAGE_ROWS, B_EDGES), jnp.int32),
            pltpu.VMEM((STAGE_ROWS, B_EDGES), jnp.float32),
            pltpu.VMEM((B_EDGES, feat), jnp.float32),
            pltpu.VMEM((B_EDGES, feat), jnp.float32),
            pltpu.VMEM_SHARED((N_PAD, feat), jnp.float32),
            pltpu.SemaphoreType.DMA,
            pltpu.SemaphoreType.DMA,
            pltpu.SemaphoreType.DMA,
            pltpu.SemaphoreType.DMA,
        ],
        compiler_params=pltpu.CompilerParams(use_tc_tiling_on_sc=False),
    )
    def spmm(sup_sr_h, sup_tg_h, ssr_h, dsr_h, wsr_h, stg_h, dtg_h, wtg_h,
             out_sr_h, out_tg_h, src_v, dst_v, w_v, rows_a, rows_b, acc,
             gsem_a, gsem_b, ssem_a, ssem_b):
        g = lax.axis_index("c")
        sid = lax.axis_index("s")
        bufs = (rows_a, rows_b)
        gsems = (gsem_a, gsem_b)
        ssems = (ssem_a, ssem_b)

        def zrow(r, c):
            for j in range(nvec):
                rows_a[r, pl.ds(j * 16, 16)] = jnp.zeros((16,), jnp.float32)
            return c
        lax.fori_loop(0, B_EDGES, zrow, 0)
        for i in range(ROWS_PER_TILE // B_EDGES):
            pltpu.sync_copy(
                rows_a, acc.at[pl.ds(sid * ROWS_PER_TILE + i * B_EDGES, B_EDGES)])
        plsc.subcore_barrier()

        def run(sup_h, s_h, d_h, w_h, out_h):
            def stage(si, c):
                sl = pl.ds(si * STAGE_ROWS, STAGE_ROWS)
                pltpu.sync_copy(s_h.at[sid, sl], src_v)
                pltpu.sync_copy(d_h.at[sid, sl], dst_v)
                pltpu.sync_copy(w_h.at[sid, sl], w_v)

                gd = {0: pltpu.async_copy(sup_h.at[src_v.at[0]], bufs[0],
                                          gsems[0])}
                sd = {}
                for r in range(STAGE_ROWS):
                    b = r % 2
                    gd[r].wait()
                    if r + 1 < STAGE_ROWS:
                        nb = (r + 1) % 2
                        if r >= 1:
                            sd[r - 1].wait()
                        gd[r + 1] = pltpu.async_copy(
                            sup_h.at[src_v.at[r + 1]], bufs[nb], gsems[nb])

                    def edge_group(gi, c3, _r=r, _b=b):
                        wvec = w_v[_r, pl.ds(gi * 16, 16)]
                        for i in range(16):
                            wv = wvec[i]
                            e = gi * 16 + i
                            for j in range(nvec):
                                fl = pl.ds(j * 16, 16)
                                bufs[_b][e, fl] = bufs[_b][e, fl] * wv
                        return c3
                    lax.fori_loop(0, B_EDGES // 16, edge_group, 0)
                    sd[r] = pltpu.async_copy(bufs[b], acc.at[dst_v.at[r]],
                                             ssems[b], add=True)
                sd[STAGE_ROWS - 2].wait()
                sd[STAGE_ROWS - 1].wait()
                return c
            lax.fori_loop(0, N_STAGES, stage, 0)
            plsc.subcore_barrier()
            pltpu.sync_copy(
                acc.at[pl.ds(sid * ROWS_PER_TILE, ROWS_PER_TILE)],
                out_h.at[pl.ds(sid * ROWS_PER_TILE, ROWS_PER_TILE)])

        @pl.when(g == 0)
        def _():
            run(sup_sr_h, ssr_h, dsr_h, wsr_h, out_sr_h)

        @pl.when(g == 1)
        def _():
            run(sup_tg_h, stg_h, dtg_h, wtg_h, out_tg_h)

    return spmm(sup_sr, sup_tg, *esr, *etg)


def kernel(edge_index_sr, edge_index_tg, edge_weight_sr, edge_weight_tg,
           attr_weight_sr, attr_weight_tg, emb_sr, emb_tg,
           W_s0, W_s1, W_a11, W_a12, W_a2):
    def prep(ei, ew):
        src = jnp.asarray(ei[0], jnp.int32).reshape(N_TILES, N_CHUNKS, B_EDGES)
        dst = jnp.asarray(ei[1], jnp.int32).reshape(N_TILES, N_CHUNKS, B_EDGES)
        w = ew.astype(jnp.float32).reshape(N_TILES, N_CHUNKS, B_EDGES)
        return (src, dst, w)

    esr = prep(edge_index_sr, edge_weight_sr)
    etg = prep(edge_index_tg, edge_weight_tg)

    s_sr = _matmul(emb_sr, W_s0)
    s_tg = _matmul(emb_tg, W_s0)
    a1_sr, a1_tg = _sc_spmm(s_sr, s_tg, esr, etg, 128)
    s2_sr = _matmul(a1_sr, W_s1, relu_in=True, bm=1024)
    s2_tg = _matmul(a1_tg, W_s1, relu_in=True, bm=1024)
    a2_sr, a2_tg = _sc_spmm(s2_sr, s2_tg, esr, etg, 128)
    sr_s = _relu_l2norm(a2_sr[:N_NODES])
    tg_s = _relu_l2norm(a2_tg[:N_NODES])

    t_sr = _matmul(attr_weight_sr, W_a11)
    t_tg = _matmul(attr_weight_tg, W_a12)
    b1_sr, b1_tg = _sc_spmm(t_sr, t_tg, esr, etg, 64)
    t2_sr = _matmul(b1_sr, W_a2, relu_in=True, bm=1024)
    t2_tg = _matmul(b1_tg, W_a2, relu_in=True, bm=1024)
    b2_sr, b2_tg = _sc_spmm(t2_sr, t2_tg, esr, etg, 64)
    sr_a = _relu(b2_sr[:N_NODES])
    tg_a = _relu(b2_tg[:N_NODES])

    return (sr_s, tg_s, sr_a, tg_a)
